# baseline (speedup 1.0000x reference)
import functools

import jax
import jax.numpy as jnp
from jax import lax
from jax.experimental import pallas as pl
from jax.experimental.pallas import tpu as pltpu
from jax.experimental.pallas import tpu_sc as plsc

D_MODEL = 1024
MAX_SEQ_LEN = 4096
BATCH = 8
ROWS = BATCH * MAX_SEQ_LEN
C = 16
NBUF = 2
NCHUNKS = ROWS // C
NW = 32
CHUNKS_PER_WORKER = NCHUNKS // NW

_mesh = plsc.VectorSubcoreMesh(core_axis_name="c", subcore_axis_name="s")


@functools.partial(
    pl.kernel,
    out_type=jax.ShapeDtypeStruct((ROWS, D_MODEL), jnp.float32),
    mesh=_mesh,
    scratch_types=(
        [pltpu.VMEM((32,), jnp.int32)]
        + [pltpu.VMEM((C,), jnp.int32) for _ in range(NBUF)]
        + [pltpu.VMEM((C, D_MODEL), jnp.float32) for _ in range(NBUF)]
        + [pltpu.VMEM((C, D_MODEL), jnp.float32)]
        + [pltpu.SemaphoreType.DMA for _ in range(2 * NBUF)]
    ),
)
def _pe_lookup(len_hbm, pe_hbm, out_hbm, len_v, *rest):
    idx = rest[:NBUF]
    dbuf = rest[NBUF:2 * NBUF]
    zbuf = rest[2 * NBUF]
    rsem = rest[2 * NBUF + 1:3 * NBUF + 1]
    wsem = rest[3 * NBUF + 1:]

    wid = lax.axis_index("s") * 2 + lax.axis_index("c")

    pltpu.sync_copy(len_hbm, len_v.at[pl.ds(0, BATCH)])

    zeros = jnp.zeros((16,), jnp.int32)
    for h in range(C // 16):
        idx[0][pl.ds(h * 16, 16)] = zeros
    pltpu.async_copy(pe_hbm.at[idx[0]], zbuf, rsem[0]).wait()

    lane = lax.iota(jnp.int32, 16)

    def chunk_params(k):
        r0 = pl.multiple_of((wid * CHUNKS_PER_WORKER + k) * C, C)
        b = r0 >> 12
        p0 = pl.multiple_of(r0 & (MAX_SEQ_LEN - 1), C)
        l = len_v[pl.ds(b, 16)][0]
        nd = jnp.clip(l - p0, 0, C)
        return r0, p0, nd

    def launch_gather(k, s):
        r0, p0, nd = chunk_params(k)

        @pl.when(nd > 0)
        def _():
            for h in range(C // 16):
                j = lane + h * 16
                idx[s][pl.ds(h * 16, 16)] = jnp.where(j < nd, p0 + 1 + j, 0)
            pltpu.async_copy(pe_hbm.at[idx[s]], dbuf[s], rsem[s])

    def launch_store(k, s):
        r0, p0, nd = chunk_params(k)

        @pl.when(nd > 0)
        def _():
            pltpu.make_async_copy(pe_hbm.at[idx[s]], dbuf[s], rsem[s]).wait()
            pltpu.async_copy(dbuf[s], out_hbm.at[pl.ds(r0, C)], wsem[s])

        @pl.when(nd == 0)
        def _():
            pltpu.async_copy(zbuf, out_hbm.at[pl.ds(r0, C)], wsem[s])

    def body(i, carry):
        for s in range(NBUF):
            k = NBUF * i + s

            @pl.when(k >= NBUF)
            def _():
                pltpu.make_async_copy(dbuf[s], out_hbm.at[pl.ds(0, C)],
                                      wsem[s]).wait()

            launch_gather(k, s)

            if s == 0:
                @pl.when(k >= 1)
                def _():
                    launch_store(k - 1, NBUF - 1)
            else:
                launch_store(k - 1, s - 1)
        return carry

    lax.fori_loop(0, CHUNKS_PER_WORKER // NBUF, body, 0)

    launch_store(CHUNKS_PER_WORKER - 1, (CHUNKS_PER_WORKER - 1) % NBUF)
    for s in range(NBUF):
        pltpu.make_async_copy(dbuf[s], out_hbm.at[pl.ds(0, C)], wsem[s]).wait()


def kernel(input_len, pe_table):
    out = _pe_lookup(input_len, pe_table)
    return out.reshape(BATCH, MAX_SEQ_LEN, D_MODEL)


# speedup vs baseline: 1.1497x; 1.1497x over previous
import functools

import jax
import jax.numpy as jnp
from jax import lax
from jax.experimental import pallas as pl
from jax.experimental.pallas import tpu as pltpu
from jax.experimental.pallas import tpu_sc as plsc

D_MODEL = 1024
MAX_SEQ_LEN = 4096
BATCH = 8
ROWS = BATCH * MAX_SEQ_LEN
C = 16
NBUF = 2
NCHUNKS = ROWS // C
NW = 32
CHUNKS_PER_WORKER = NCHUNKS // NW

_mesh = plsc.VectorSubcoreMesh(core_axis_name="c", subcore_axis_name="s")


@functools.partial(
    pl.kernel,
    out_type=jax.ShapeDtypeStruct((ROWS, D_MODEL), jnp.float32),
    mesh=_mesh,
    scratch_types=(
        [pltpu.VMEM((32,), jnp.int32)]
        + [pltpu.VMEM((C,), jnp.int32) for _ in range(NBUF)]
        + [pltpu.VMEM((C, D_MODEL), jnp.float32) for _ in range(NBUF)]
        + [pltpu.VMEM((C, D_MODEL), jnp.float32)]
        + [pltpu.SemaphoreType.DMA for _ in range(2 * NBUF)]
    ),
)
def _pe_lookup(len_hbm, pe_hbm, out_hbm, len_v, *rest):
    idx = rest[:NBUF]
    dbuf = rest[NBUF:2 * NBUF]
    zbuf = rest[2 * NBUF]
    rsem = rest[2 * NBUF + 1:3 * NBUF + 1]
    wsem = rest[3 * NBUF + 1:]

    wid = lax.axis_index("c") * 16 + lax.axis_index("s")

    pltpu.sync_copy(len_hbm, len_v.at[pl.ds(0, BATCH)])

    zeros = jnp.zeros((16,), jnp.int32)
    for h in range(C // 16):
        idx[0][pl.ds(h * 16, 16)] = zeros
    pltpu.async_copy(pe_hbm.at[idx[0]], zbuf, rsem[0]).wait()

    lane = lax.iota(jnp.int32, 16)

    def chunk_params(k):
        r0 = pl.multiple_of((wid + k * NW) * C, C)
        b = r0 >> 12
        p0 = pl.multiple_of(r0 & (MAX_SEQ_LEN - 1), C)
        l = len_v[pl.ds(b, 16)][0]
        nd = jnp.clip(l - p0, 0, C)
        return r0, p0, nd

    def launch_gather(k, s):
        r0, p0, nd = chunk_params(k)

        @pl.when(nd > 0)
        def _():
            for h in range(C // 16):
                j = lane + h * 16
                idx[s][pl.ds(h * 16, 16)] = jnp.where(j < nd, p0 + 1 + j, 0)
            pltpu.async_copy(pe_hbm.at[idx[s]], dbuf[s], rsem[s])

    def launch_store(k, s):
        r0, p0, nd = chunk_params(k)

        @pl.when(nd > 0)
        def _():
            pltpu.make_async_copy(pe_hbm.at[idx[s]], dbuf[s], rsem[s]).wait()
            pltpu.async_copy(dbuf[s], out_hbm.at[pl.ds(r0, C)], wsem[s])

        @pl.when(nd == 0)
        def _():
            pltpu.async_copy(zbuf, out_hbm.at[pl.ds(r0, C)], wsem[s])

    def body(i, carry):
        for s in range(NBUF):
            k = NBUF * i + s

            @pl.when(k >= NBUF)
            def _():
                pltpu.make_async_copy(dbuf[s], out_hbm.at[pl.ds(0, C)],
                                      wsem[s]).wait()

            launch_gather(k, s)

            if s == 0:
                @pl.when(k >= 1)
                def _():
                    launch_store(k - 1, NBUF - 1)
            else:
                launch_store(k - 1, s - 1)
        return carry

    lax.fori_loop(0, CHUNKS_PER_WORKER // NBUF, body, 0)

    launch_store(CHUNKS_PER_WORKER - 1, (CHUNKS_PER_WORKER - 1) % NBUF)
    for s in range(NBUF):
        pltpu.make_async_copy(dbuf[s], out_hbm.at[pl.ds(0, C)], wsem[s]).wait()


def kernel(input_len, pe_table):
    out = _pe_lookup(input_len, pe_table)
    return out.reshape(BATCH, MAX_SEQ_LEN, D_MODEL)


# speedup vs baseline: 1.1624x; 1.0111x over previous
import functools

import jax
import jax.numpy as jnp
from jax import lax
from jax.experimental import pallas as pl
from jax.experimental.pallas import tpu as pltpu
from jax.experimental.pallas import tpu_sc as plsc

D_MODEL = 1024
MAX_SEQ_LEN = 4096
BATCH = 8
ROWS = BATCH * MAX_SEQ_LEN
C = 16
NBUF = 2
NCHUNKS = ROWS // C
NW = 32
CHUNKS_PER_WORKER = NCHUNKS // NW

_mesh = plsc.VectorSubcoreMesh(core_axis_name="c", subcore_axis_name="s")


@functools.partial(
    pl.kernel,
    out_type=jax.ShapeDtypeStruct((ROWS, D_MODEL), jnp.float32),
    mesh=_mesh,
    scratch_types=(
        [pltpu.VMEM((32,), jnp.int32)]
        + [pltpu.VMEM((C,), jnp.int32) for _ in range(NBUF)]
        + [pltpu.VMEM((C, D_MODEL), jnp.float32) for _ in range(NBUF)]
        + [pltpu.VMEM((C, D_MODEL), jnp.float32)]
        + [pltpu.SemaphoreType.DMA for _ in range(2 * NBUF)]
    ),
)
def _pe_lookup(len_hbm, pe_hbm, out_hbm, len_v, *rest):
    idx = rest[:NBUF]
    dbuf = rest[NBUF:2 * NBUF]
    zbuf = rest[2 * NBUF]
    rsem = rest[2 * NBUF + 1:3 * NBUF + 1]
    wsem = rest[3 * NBUF + 1:]

    wid = lax.axis_index("s") * 2 + lax.axis_index("c")

    pltpu.sync_copy(len_hbm, len_v.at[pl.ds(0, BATCH)])

    zeros = jnp.zeros((16,), jnp.int32)
    for h in range(C // 16):
        idx[0][pl.ds(h * 16, 16)] = zeros
    pltpu.async_copy(pe_hbm.at[idx[0]], zbuf, rsem[0]).wait()

    lane = lax.iota(jnp.int32, 16)

    def chunk_params(k):
        r0 = pl.multiple_of((wid + k * NW) * C, C)
        b = r0 >> 12
        p0 = pl.multiple_of(r0 & (MAX_SEQ_LEN - 1), C)
        l = len_v[pl.ds(b, 16)][0]
        nd = jnp.clip(l - p0, 0, C)
        return r0, p0, nd

    def launch_gather(k, s):
        r0, p0, nd = chunk_params(k)

        @pl.when(nd > 0)
        def _():
            for h in range(C // 16):
                j = lane + h * 16
                idx[s][pl.ds(h * 16, 16)] = jnp.where(j < nd, p0 + 1 + j, 0)
            pltpu.async_copy(pe_hbm.at[idx[s]], dbuf[s], rsem[s])

    def launch_store(k, s):
        r0, p0, nd = chunk_params(k)

        @pl.when(nd > 0)
        def _():
            pltpu.make_async_copy(pe_hbm.at[idx[s]], dbuf[s], rsem[s]).wait()
            pltpu.async_copy(dbuf[s], out_hbm.at[pl.ds(r0, C)], wsem[s])

        @pl.when(nd == 0)
        def _():
            pltpu.async_copy(zbuf, out_hbm.at[pl.ds(r0, C)], wsem[s])

    def body(i, carry):
        for s in range(NBUF):
            k = NBUF * i + s

            @pl.when(k >= NBUF)
            def _():
                pltpu.make_async_copy(dbuf[s], out_hbm.at[pl.ds(0, C)],
                                      wsem[s]).wait()

            launch_gather(k, s)

            if s == 0:
                @pl.when(k >= 1)
                def _():
                    launch_store(k - 1, NBUF - 1)
            else:
                launch_store(k - 1, s - 1)
        return carry

    lax.fori_loop(0, CHUNKS_PER_WORKER // NBUF, body, 0)

    launch_store(CHUNKS_PER_WORKER - 1, (CHUNKS_PER_WORKER - 1) % NBUF)
    for s in range(NBUF):
        pltpu.make_async_copy(dbuf[s], out_hbm.at[pl.ds(0, C)], wsem[s]).wait()


def kernel(input_len, pe_table):
    out = _pe_lookup(input_len, pe_table)
    return out.reshape(BATCH, MAX_SEQ_LEN, D_MODEL)
